# Initial kernel scaffold; baseline (speedup 1.0000x reference)
import jax
import jax.numpy as jnp
from jax.experimental import pallas as pl


def kernel(xyz, features, params):
    raise NotImplementedError("write your pallas kernel here")


# speedup vs baseline: 6.1446x; 6.1446x over previous
import functools

import jax
import jax.numpy as jnp
from jax.experimental import pallas as pl

_SA_SPECS = [(1024, 0.1, 64), (256, 0.2, 32), (64, 0.4, 16), (16, 0.8, 8)]
_SA_MLPS = [[6, 32, 64], [64, 64, 128], [128, 128, 256], [256, 256, 512]]
_FP_MLPS = [[768, 512, 256], [384, 256, 128], [192, 128, 64], [70, 32, 16, 64]]


def _fps_body(xyz_ref, xyzT_ref, out_ref, *, npoint, n, nb):
    iota = jax.lax.broadcasted_iota(jnp.int32, (1, n), 1).astype(jnp.float32)
    xyzTs = [xyzT_ref[b] for b in range(nb)]
    for b in range(nb):
        out_ref[b, 0:1, :] = xyz_ref[b, 0:1, :]

    def body(i, carry):
        ds = list(carry[:nb])
        ls = list(carry[nb:])
        for b in range(nb):
            diff = xyzTs[b] - ls[b]
            d = jnp.sum(diff * diff, axis=0, keepdims=True)
            nd = jnp.minimum(ds[b], d)
            m = jnp.max(nd, axis=1, keepdims=True)
            cand = jnp.where(nd == m, iota, float(n))
            nxt = jnp.min(cand, axis=1, keepdims=True)
            oh = (iota == nxt).astype(jnp.float32)
            last_t = jnp.sum(xyzTs[b] * oh, axis=1, keepdims=True)
            last_row = jnp.concatenate(
                [last_t[0:1, :], last_t[1:2, :], last_t[2:3, :]], axis=1)
            out_ref[b, pl.ds(i, 1), :] = last_row
            ds[b] = nd
            ls[b] = last_t
        return tuple(ds) + tuple(ls)

    init = tuple(jnp.full((1, n), 1e10, jnp.float32) for _ in range(nb)) + \
        tuple(xyzTs[b][:, 0:1] for b in range(nb))
    jax.lax.fori_loop(1, npoint, body, init)


def _fps(xyz, xyzT, npoint):
    nb, n, _ = xyz.shape
    return pl.pallas_call(
        functools.partial(_fps_body, npoint=npoint, n=n, nb=nb),
        out_shape=jax.ShapeDtypeStruct((nb, npoint, 3), jnp.float32),
    )(xyz, xyzT)


def _mlp_body(f_ref, w1_ref, b1_ref, w2_ref, b2_ref, out_ref):
    x = f_ref[0]
    h = jnp.maximum(
        jnp.dot(x, w1_ref[...], preferred_element_type=jnp.float32)
        + b1_ref[...], 0.0)
    g = jnp.maximum(
        jnp.dot(h, w2_ref[...], preferred_element_type=jnp.float32)
        + b2_ref[...], 0.0)
    out_ref[0] = g


def _sa_mlp(fT, w1, b1, w2, b2):
    nb, n, c = fT.shape
    c2 = w2.shape[1]
    full = lambda shape: pl.BlockSpec(shape, lambda b: (0, 0))
    return pl.pallas_call(
        _mlp_body,
        grid=(nb,),
        in_specs=[
            pl.BlockSpec((1, n, c), lambda b: (b, 0, 0)),
            full(w1.shape), full((1, b1.shape[1])),
            full(w2.shape), full((1, b2.shape[1])),
        ],
        out_specs=pl.BlockSpec((1, n, c2), lambda b: (b, 0, 0)),
        out_shape=jax.ShapeDtypeStruct((nb, n, c2), jnp.float32),
    )(fT, w1, b1, w2, b2)


def _pool_body(nxT_ref, xyz_ref, g_ref, out_ref, *, r2, ns, n, ck, c2, sblk):
    rk = 8
    aT = nxT_ref[0]
    aa = jnp.sum(aT * aT, axis=0, keepdims=True)
    ri = jax.lax.broadcasted_iota(jnp.int32, (ck, ck), 0)
    cj = jax.lax.broadcasted_iota(jnp.int32, (ck, ck), 1)
    low = (cj <= ri).astype(jnp.float32)

    def chunk(c, state):
        carry, acc = state
        k0 = c * ck
        xc = xyz_ref[0, pl.ds(k0, ck), :]
        gc = g_ref[0, pl.ds(k0, ck), :]
        ppc = jnp.sum(xc * xc, axis=1, keepdims=True)
        cross = jnp.dot(xc.astype(jnp.bfloat16), aT.astype(jnp.bfloat16),
                        preferred_element_type=jnp.float32)
        d2t = jnp.maximum(aa + ppc - 2.0 * cross, 0.0)
        mct = (d2t < r2).astype(jnp.float32)
        cst = jnp.dot(low, mct, preferred_element_type=jnp.float32) + carry
        m2t = mct * (cst <= float(ns)).astype(jnp.float32)
        for r in range(ck // rk):
            m2r = m2t[r * rk:(r + 1) * rk, :]
            gr = gc[r * rk:(r + 1) * rk, :]
            t = m2r[:, :, None] * gr[:, None, :]
            acc = jnp.maximum(acc, jnp.max(t, axis=0))
        return (cst[ck - 1:ck, :], acc)

    init = (jnp.zeros((1, sblk), jnp.float32), jnp.zeros((sblk, c2), jnp.float32))
    _, acc = jax.lax.fori_loop(0, n // ck, chunk, init)
    out_ref[0] = acc


def _sa_pool(new_xyzT, xyz, g, radius, ns):
    nb, _, s = new_xyzT.shape
    n = xyz.shape[1]
    c2 = g.shape[2]
    sblk = min(s, 128)
    ck = min(n, 128)
    body = functools.partial(_pool_body, r2=radius * radius, ns=ns, n=n,
                             ck=ck, c2=c2, sblk=sblk)
    return pl.pallas_call(
        body,
        grid=(nb, s // sblk),
        in_specs=[
            pl.BlockSpec((1, 3, sblk), lambda b, i: (b, 0, i)),
            pl.BlockSpec((1, n, 3), lambda b, i: (b, 0, 0)),
            pl.BlockSpec((1, n, c2), lambda b, i: (b, 0, 0)),
        ],
        out_specs=pl.BlockSpec((1, sblk, c2), lambda b, i: (b, i, 0)),
        out_shape=jax.ShapeDtypeStruct((nb, s, c2), jnp.float32),
    )(new_xyzT, xyz, g)


def _fp_body(*refs, s, nl):
    tp = refs[0][0]
    srcT = refs[1][0]
    fe = refs[2][0]
    tf = refs[3][0]
    out_ref = refs[-1]
    tt = jnp.sum(tp * tp, axis=1, keepdims=True)
    ss = jnp.sum(srcT * srcT, axis=0, keepdims=True)
    d2 = jnp.maximum(
        tt + ss - 2.0 * jnp.dot(tp.astype(jnp.bfloat16),
                                srcT.astype(jnp.bfloat16),
                                preferred_element_type=jnp.float32),
        0.0)
    iota = jax.lax.broadcasted_iota(jnp.int32, (1, s), 1).astype(jnp.float32)
    amat = jnp.zeros(d2.shape, jnp.float32)
    wsum = jnp.zeros((d2.shape[0], 1), jnp.float32)
    for _ in range(3):
        v = jnp.min(d2, axis=1, keepdims=True)
        cand = jnp.where(d2 == v, iota, float(s))
        i1 = jnp.min(cand, axis=1, keepdims=True)
        oh = (iota == i1).astype(jnp.float32)
        w = 1.0 / (v + 1e-8)
        amat = amat + w * oh
        wsum = wsum + w
        d2 = jnp.where(oh > 0.0, 1e30, d2)
    amat = amat / wsum
    interp = jnp.dot(amat, fe, preferred_element_type=jnp.float32)
    wa, wb, b1 = refs[4][...], refs[5][...], refs[6][...]
    h = jnp.maximum(
        jnp.dot(interp, wa, preferred_element_type=jnp.float32)
        + jnp.dot(tf, wb, preferred_element_type=jnp.float32) + b1, 0.0)
    for l in range(nl - 1):
        w_r = refs[7 + 2 * l][...]
        b_r = refs[8 + 2 * l][...]
        h = jnp.maximum(
            jnp.dot(h, w_r, preferred_element_type=jnp.float32) + b_r, 0.0)
    out_ref[0] = h


def _fp(tp, srcT, feats, tfT, layers):
    nb, nt, _ = tp.shape
    s = srcT.shape[2]
    ci = feats.shape[2]
    cs = tfT.shape[2]
    cout = layers[-1][0].shape[1]
    tblk = min(nt, 512)
    w1, bb1 = layers[0]
    wa, wb = w1[:ci], w1[ci:]
    wargs = [wa, wb, bb1.reshape(1, -1)]
    for (w, b) in layers[1:]:
        wargs += [w, b.reshape(1, -1)]
    full2 = lambda arr: pl.BlockSpec(arr.shape, lambda b, i: (0, 0))
    body = functools.partial(_fp_body, s=s, nl=len(layers))
    return pl.pallas_call(
        body,
        grid=(nb, nt // tblk),
        in_specs=[
            pl.BlockSpec((1, tblk, 3), lambda b, i: (b, i, 0)),
            pl.BlockSpec((1, 3, s), lambda b, i: (b, 0, 0)),
            pl.BlockSpec((1, s, ci), lambda b, i: (b, 0, 0)),
            pl.BlockSpec((1, tblk, cs), lambda b, i: (b, i, 0)),
        ] + [full2(a) for a in wargs],
        out_specs=pl.BlockSpec((1, tblk, cout), lambda b, i: (b, i, 0)),
        out_shape=jax.ShapeDtypeStruct((nb, nt, cout), jnp.float32),
    )(tp, srcT, feats, tfT, *wargs)


def _split(params):
    pi = 0
    sa_p, fp_p = [], []
    for mlp in _SA_MLPS:
        k = len(mlp) - 1
        sa_p.append(params[pi:pi + k])
        pi += k
    for mlp in _FP_MLPS:
        k = len(mlp) - 1
        fp_p.append(params[pi:pi + k])
        pi += k
    return sa_p, fp_p


@jax.jit
def kernel(xyz, features, params):
    sa_p, fp_p = _split(params)
    points = xyz
    fT = features
    sa_pts, sa_fts = [], []
    for i, (np_, rad, ns) in enumerate(_SA_SPECS):
        sa_pts.append(points)
        sa_fts.append(fT)
        pointsT = jnp.transpose(points, (0, 2, 1))
        new_xyz = _fps(points, pointsT, np_)
        new_xyzT = jnp.transpose(new_xyz, (0, 2, 1))
        (w1, b1), (w2, b2) = sa_p[i]
        g = _sa_mlp(fT, w1, b1.reshape(1, -1), w2, b2.reshape(1, -1))
        fT = _sa_pool(new_xyzT, points, g, rad, ns)
        points = new_xyz
    for j in range(4):
        tp = sa_pts[-(j + 1)]
        tf = sa_fts[-(j + 1)]
        srcT = jnp.transpose(points, (0, 2, 1))
        fT = _fp(tp, srcT, fT, tf, fp_p[j])
        points = tp
    return (xyz, fT)
